# baseline (speedup 1.0000x reference)
import functools

import jax
import jax.numpy as jnp
from jax import lax
from jax.experimental import pallas as pl
from jax.experimental.pallas import tpu as pltpu
from jax.experimental.pallas import tpu_sc as plsc

B, N, C, K = 4, 1024, 256, 16
QB = 128
NQB = N // QB
TW = 2 * C
PW = 128

SC_CORES, SC_SUBCORES = 2, 16
NW = SC_CORES * SC_SUBCORES
ROWS = N * K
RPW = ROWS // NW
CH = 64
NCH = RPW // CH

_f32 = jnp.float32
_bf16 = jnp.bfloat16


def _prep_body(pos_ref, posT_ref, feat_ref, W_emb_ref, b_emb_ref,
               W_q_ref, W_k_ref, W_v_ref,
               q_ref, table_ref, ptab_ref, knn_ref):
    pos = pos_ref[0]
    posT = posT_ref[0]
    x = jnp.dot(feat_ref[0], W_emb_ref[...],
                preferred_element_type=_f32) + b_emb_ref[...]
    q_ref[0] = jnp.dot(x, W_q_ref[...], preferred_element_type=_f32)
    xk = jnp.dot(x, W_k_ref[...], preferred_element_type=_f32)
    xv = jnp.dot(x, W_v_ref[...], preferred_element_type=_f32)
    table_ref[0] = jnp.concatenate([xk, xv], axis=1).astype(_bf16)
    ptab_ref[0] = jnp.concatenate([pos, jnp.zeros((N, PW - 3), _f32)], axis=1)

    d2 = (pos[:, 0:1] - posT[0:1, :]) ** 2
    d2 = d2 + (pos[:, 1:2] - posT[1:2, :]) ** 2
    d2 = d2 + (pos[:, 2:3] - posT[2:3, :]) ** 2

    iota = lax.broadcasted_iota(jnp.int32, (N, N), 1)
    work = d2
    cols = []
    for _ in range(K):
        m = jnp.min(work, axis=1, keepdims=True)
        idx = jnp.min(jnp.where(work == m, iota, N), axis=1, keepdims=True)
        cols.append(idx)
        work = jnp.where(iota == idx, jnp.inf, work)
    knn_ref[0] = jnp.concatenate(cols, axis=1)


def _gather_body(table_hbm, ptab_hbm, idx_hbm, out_hbm, outp_hbm,
                 idx_v, bufk0, bufk1, bufp0, bufp1,
                 semk0, semk1, semp0, semp1):
    wid = lax.axis_index("s") * SC_CORES + lax.axis_index("c")
    base = wid * RPW
    pltpu.sync_copy(idx_hbm.at[pl.ds(base, RPW)], idx_v)
    bufk = (bufk0, bufk1)
    bufp = (bufp0, bufp1)
    semk = (semk0, semk1)
    semp = (semp0, semp1)

    def start(ch, slot):
        i = idx_v.at[pl.ds(ch * CH, CH)]
        return (pltpu.async_copy(table_hbm.at[i], bufk[slot], semk[slot]),
                pltpu.async_copy(ptab_hbm.at[i], bufp[slot], semp[slot]))

    desc = [None] * NCH
    desc[0] = start(0, 0)
    if NCH > 1:
        desc[1] = start(1, 1)
    for ch in range(NCH):
        slot = ch % 2
        desc[ch][0].wait()
        desc[ch][1].wait()
        pltpu.sync_copy(bufk[slot], out_hbm.at[pl.ds(base + ch * CH, CH)])
        pltpu.sync_copy(bufp[slot], outp_hbm.at[pl.ds(base + ch * CH, CH)])
        if ch + 2 < NCH:
            desc[ch + 2] = start(ch + 2, slot)


def _attn_body(g_ref, gp_ref, q_ref, posq_ref, feat_ref,
               Wp1_ref, bp1_ref, Wp2_ref, bp2_ref,
               Wa1_ref, ba1_ref, Wa2_ref, ba2_ref,
               W_out_ref, b_out_ref, out_ref):
    g = g_ref[0]
    kk = g[:, 0:C].astype(_f32)
    v = g[:, C:2 * C].astype(_f32)
    gpos = gp_ref[0][:, 0:3]

    posq = posq_ref[0]
    posq_rep = jnp.broadcast_to(posq[:, None, :], (QB, K, 3)).reshape(QB * K, 3)
    gpos = gpos - posq_rep

    h = jnp.maximum(
        jnp.dot(gpos, Wp1_ref[...], preferred_element_type=_f32) + bp1_ref[...],
        0.0)
    posenc = jnp.dot(h.astype(_bf16), Wp2_ref[...],
                     preferred_element_type=_f32) + bp2_ref[...]

    q = q_ref[0]
    q_rep = jnp.broadcast_to(q[:, None, :], (QB, K, C)).reshape(QB * K, C)
    pre = q_rep - kk + posenc
    h2 = jnp.maximum(
        jnp.dot(pre.astype(_bf16), Wa1_ref[...],
                preferred_element_type=_f32) + ba1_ref[...],
        0.0)
    attn = jnp.dot(h2.astype(_bf16), Wa2_ref[...],
                   preferred_element_type=_f32) + ba2_ref[...]

    s = (attn * (1.0 / 16.0)).reshape(QB, K, C)
    m = jnp.max(s, axis=1, keepdims=True)
    e = jnp.exp(s - m)
    w = e / jnp.sum(e, axis=1, keepdims=True)

    vp = (v + posenc).reshape(QB, K, C)
    res = jnp.sum(w * vp, axis=1)
    out_ref[0] = (jnp.dot(res.astype(_bf16), W_out_ref[...],
                          preferred_element_type=_f32)
                  + b_out_ref[...] + feat_ref[0])


def _full(shape):
    return pl.BlockSpec(shape, lambda *args: tuple(0 for _ in shape))


def kernel(pos, features, W_emb, b_emb, W_q, W_k, W_v, Wp1, bp1, Wp2, bp2,
           Wa1, ba1, Wa2, ba2, W_out, b_out):
    posT = jnp.transpose(pos, (0, 2, 1))
    b_emb2 = b_emb.reshape(1, C)
    bp12 = bp1.reshape(1, C)
    bp22 = bp2.reshape(1, C)
    ba12 = ba1.reshape(1, C)
    ba22 = ba2.reshape(1, C)
    b_out2 = b_out.reshape(1, -1)
    Wp2b = Wp2.astype(_bf16)
    Wa1b = Wa1.astype(_bf16)
    Wa2b = Wa2.astype(_bf16)
    W_outb = W_out.astype(_bf16)

    prep = pl.pallas_call(
        _prep_body,
        grid=(1,),
        in_specs=[
            pl.BlockSpec((1, N, 3), lambda i: (0, 0, 0)),
            pl.BlockSpec((1, 3, N), lambda i: (0, 0, 0)),
            pl.BlockSpec((1, N, C), lambda i: (0, 0, 0)),
            _full(W_emb.shape), _full((1, C)),
            _full(W_q.shape), _full(W_k.shape), _full(W_v.shape),
        ],
        out_specs=[
            pl.BlockSpec((1, N, C), lambda i: (0, 0, 0)),
            pl.BlockSpec((1, N, TW), lambda i: (0, 0, 0)),
            pl.BlockSpec((1, N, PW), lambda i: (0, 0, 0)),
            pl.BlockSpec((1, N, K), lambda i: (0, 0, 0)),
        ],
        out_shape=[
            jax.ShapeDtypeStruct((1, N, C), _f32),
            jax.ShapeDtypeStruct((1, N, TW), _bf16),
            jax.ShapeDtypeStruct((1, N, PW), _f32),
            jax.ShapeDtypeStruct((1, N, K), jnp.int32),
        ],
    )

    sc_gather = pl.kernel(
        _gather_body,
        out_type=[
            jax.ShapeDtypeStruct((ROWS, TW // 2), jnp.int32),
            jax.ShapeDtypeStruct((ROWS, PW), _f32),
        ],
        mesh=plsc.VectorSubcoreMesh(core_axis_name="c", subcore_axis_name="s",
                                    num_cores=SC_CORES),
        scratch_types=[
            pltpu.VMEM((RPW,), jnp.int32),
            pltpu.VMEM((CH, TW // 2), jnp.int32),
            pltpu.VMEM((CH, TW // 2), jnp.int32),
            pltpu.VMEM((CH, PW), _f32),
            pltpu.VMEM((CH, PW), _f32),
            pltpu.SemaphoreType.DMA,
            pltpu.SemaphoreType.DMA,
            pltpu.SemaphoreType.DMA,
            pltpu.SemaphoreType.DMA,
        ],
    )

    attn = pl.pallas_call(
        _attn_body,
        grid=(NQB,),
        in_specs=[
            pl.BlockSpec((1, QB * K, TW), lambda qb: (0, qb, 0)),
            pl.BlockSpec((1, QB * K, PW), lambda qb: (0, qb, 0)),
            pl.BlockSpec((1, QB, C), lambda qb: (0, qb, 0)),
            pl.BlockSpec((1, QB, 3), lambda qb: (0, qb, 0)),
            pl.BlockSpec((1, QB, C), lambda qb: (0, qb, 0)),
            _full(Wp1.shape), _full((1, C)),
            _full(Wp2.shape), _full((1, C)),
            _full(Wa1.shape), _full((1, C)),
            _full(Wa2.shape), _full((1, C)),
            _full(W_out.shape), _full((1, b_out.shape[0])),
        ],
        out_specs=pl.BlockSpec((1, QB, C), lambda qb: (0, qb, 0)),
        out_shape=jax.ShapeDtypeStruct((1, N, C), _f32),
    )

    outs = []
    for b in range(B):
        q, table, ptab, knn = prep(pos[b:b + 1], posT[b:b + 1],
                                   features[b:b + 1], W_emb, b_emb2,
                                   W_q, W_k, W_v)
        tab_i32 = lax.bitcast_convert_type(
            table.reshape(N, TW // 2, 2), jnp.int32)
        g_i32, gp = sc_gather(tab_i32, ptab.reshape(N, PW),
                              knn.reshape(ROWS))
        g = lax.bitcast_convert_type(g_i32, _bf16).reshape(ROWS, TW)
        outs.append(attn(g.reshape(1, N * K, TW), gp.reshape(1, N * K, PW),
                         q, pos[b:b + 1], features[b:b + 1],
                         Wp1, bp12, Wp2b, bp22, Wa1b, ba12, Wa2b, ba22,
                         W_outb, b_out2))
    return jnp.concatenate(outs, axis=0)


# speedup vs baseline: 3.0673x; 3.0673x over previous
import functools

import jax
import jax.numpy as jnp
from jax import lax
from jax.experimental import pallas as pl
from jax.experimental.pallas import tpu as pltpu
from jax.experimental.pallas import tpu_sc as plsc

B, N, C, K = 4, 1024, 256, 16
QB = 128
NQB = N // QB
TW = C + 128

SC_CORES, SC_SUBCORES = 2, 16
NW = SC_CORES * SC_SUBCORES
ROWS = N * K
RPW = ROWS // NW
CH = 128
NCH = RPW // CH

_f32 = jnp.float32
_bf16 = jnp.bfloat16
_HI = -65536


def _rne_hi(u):
    return (u + 0x7FFF + (lax.shift_right_logical(u, 16) & 1)) & _HI


def _prep_body(pos_ref, posT_ref, feat_ref, W_emb_ref, b_emb_ref,
               W_q_ref, W_k_ref, W_v_ref,
               q_ref, table_ref, knn_ref):
    pos = pos_ref[0]
    posT = posT_ref[0]
    x = jnp.dot(feat_ref[0], W_emb_ref[...],
                preferred_element_type=_f32) + b_emb_ref[...]
    q_ref[0] = jnp.dot(x, W_q_ref[...], preferred_element_type=_f32)
    xk = jnp.dot(x, W_k_ref[...], preferred_element_type=_f32)
    xv = jnp.dot(x, W_v_ref[...], preferred_element_type=_f32)

    uk = lax.bitcast_convert_type(xk, jnp.int32)
    uv = lax.bitcast_convert_type(xv, jnp.int32)
    packed = lax.shift_right_logical(_rne_hi(uk), 16) | _rne_hi(uv)
    pos_pad = jnp.concatenate([pos, jnp.zeros((N, 125), _f32)], axis=1)
    table_ref[0] = jnp.concatenate(
        [packed, lax.bitcast_convert_type(pos_pad, jnp.int32)], axis=1)

    d2 = (pos[:, 0:1] - posT[0:1, :]) ** 2
    d2 = d2 + (pos[:, 1:2] - posT[1:2, :]) ** 2
    d2 = d2 + (pos[:, 2:3] - posT[2:3, :]) ** 2

    iota = lax.broadcasted_iota(jnp.int32, (N, N), 1)
    work = d2
    cols = []
    for _ in range(K):
        m = jnp.min(work, axis=1, keepdims=True)
        idx = jnp.min(jnp.where(work == m, iota, N), axis=1, keepdims=True)
        cols.append(idx)
        work = jnp.where(iota == idx, jnp.inf, work)
    knn_ref[0] = jnp.concatenate(cols, axis=1)


def _gather_body(table_hbm, idx_hbm, out_hbm, idx_v, buf0, buf1, sem0, sem1):
    wid = lax.axis_index("s") * SC_CORES + lax.axis_index("c")
    base = wid * RPW
    pltpu.sync_copy(idx_hbm.at[pl.ds(base, RPW)], idx_v)
    bufs = (buf0, buf1)
    sems = (sem0, sem1)

    def start(ch, slot):
        return pltpu.async_copy(
            table_hbm.at[idx_v.at[pl.ds(ch * CH, CH)]], bufs[slot], sems[slot])

    desc = [None] * NCH
    desc[0] = start(0, 0)
    if NCH > 1:
        desc[1] = start(1, 1)
    for ch in range(NCH):
        slot = ch % 2
        desc[ch].wait()
        pltpu.sync_copy(bufs[slot], out_hbm.at[pl.ds(base + ch * CH, CH)])
        if ch + 2 < NCH:
            desc[ch + 2] = start(ch + 2, slot)


def _attn_body(g_ref, q_ref, posq_ref, feat_ref,
               Wp1_ref, bp1_ref, Wp2_ref, bp2_ref,
               Wa1_ref, ba1_ref, Wa2_ref, ba2_ref,
               W_out_ref, b_out_ref, out_ref):
    gi = g_ref[0]
    kk = lax.bitcast_convert_type(lax.shift_left(gi[:, 0:C], 16), _f32)
    v = lax.bitcast_convert_type(gi[:, 0:C] & _HI, _f32)
    gpos = lax.bitcast_convert_type(gi[:, C:C + 3], _f32)

    posq = posq_ref[0]
    posq_rep = jnp.broadcast_to(posq[:, None, :], (QB, K, 3)).reshape(QB * K, 3)
    gpos = gpos - posq_rep

    h = jnp.maximum(
        jnp.dot(gpos, Wp1_ref[...], preferred_element_type=_f32) + bp1_ref[...],
        0.0)
    posenc = jnp.dot(h.astype(_bf16), Wp2_ref[...],
                     preferred_element_type=_f32) + bp2_ref[...]

    q = q_ref[0]
    q_rep = jnp.broadcast_to(q[:, None, :], (QB, K, C)).reshape(QB * K, C)
    pre = q_rep - kk + posenc
    h2 = jnp.maximum(
        jnp.dot(pre.astype(_bf16), Wa1_ref[...],
                preferred_element_type=_f32) + ba1_ref[...],
        0.0)
    attn = jnp.dot(h2.astype(_bf16), Wa2_ref[...],
                   preferred_element_type=_f32) + ba2_ref[...]

    s = (attn * (1.0 / 16.0)).reshape(QB, K, C)
    m = jnp.max(s, axis=1, keepdims=True)
    e = jnp.exp(s - m)
    w = e / jnp.sum(e, axis=1, keepdims=True)

    vp = (v + posenc).reshape(QB, K, C)
    res = jnp.sum(w * vp, axis=1)
    out_ref[0] = (jnp.dot(res.astype(_bf16), W_out_ref[...],
                          preferred_element_type=_f32)
                  + b_out_ref[...] + feat_ref[0])


def _full(shape):
    return pl.BlockSpec(shape, lambda *args: tuple(0 for _ in shape))


def kernel(pos, features, W_emb, b_emb, W_q, W_k, W_v, Wp1, bp1, Wp2, bp2,
           Wa1, ba1, Wa2, ba2, W_out, b_out):
    posT = jnp.transpose(pos, (0, 2, 1))
    b_emb2 = b_emb.reshape(1, C)
    bp12 = bp1.reshape(1, C)
    bp22 = bp2.reshape(1, C)
    ba12 = ba1.reshape(1, C)
    ba22 = ba2.reshape(1, C)
    b_out2 = b_out.reshape(1, -1)
    Wp2b = Wp2.astype(_bf16)
    Wa1b = Wa1.astype(_bf16)
    Wa2b = Wa2.astype(_bf16)
    W_outb = W_out.astype(_bf16)

    prep = pl.pallas_call(
        _prep_body,
        grid=(1,),
        in_specs=[
            pl.BlockSpec((1, N, 3), lambda i: (0, 0, 0)),
            pl.BlockSpec((1, 3, N), lambda i: (0, 0, 0)),
            pl.BlockSpec((1, N, C), lambda i: (0, 0, 0)),
            _full(W_emb.shape), _full((1, C)),
            _full(W_q.shape), _full(W_k.shape), _full(W_v.shape),
        ],
        out_specs=[
            pl.BlockSpec((1, N, C), lambda i: (0, 0, 0)),
            pl.BlockSpec((1, N, TW), lambda i: (0, 0, 0)),
            pl.BlockSpec((1, N, K), lambda i: (0, 0, 0)),
        ],
        out_shape=[
            jax.ShapeDtypeStruct((1, N, C), _f32),
            jax.ShapeDtypeStruct((1, N, TW), jnp.int32),
            jax.ShapeDtypeStruct((1, N, K), jnp.int32),
        ],
    )

    sc_gather = pl.kernel(
        _gather_body,
        out_type=jax.ShapeDtypeStruct((ROWS, TW), jnp.int32),
        mesh=plsc.VectorSubcoreMesh(core_axis_name="c", subcore_axis_name="s",
                                    num_cores=SC_CORES),
        scratch_types=[
            pltpu.VMEM((RPW,), jnp.int32),
            pltpu.VMEM((CH, TW), jnp.int32),
            pltpu.VMEM((CH, TW), jnp.int32),
            pltpu.SemaphoreType.DMA,
            pltpu.SemaphoreType.DMA,
        ],
    )

    attn = pl.pallas_call(
        _attn_body,
        grid=(NQB,),
        in_specs=[
            pl.BlockSpec((1, QB * K, TW), lambda qb: (0, qb, 0)),
            pl.BlockSpec((1, QB, C), lambda qb: (0, qb, 0)),
            pl.BlockSpec((1, QB, 3), lambda qb: (0, qb, 0)),
            pl.BlockSpec((1, QB, C), lambda qb: (0, qb, 0)),
            _full(Wp1.shape), _full((1, C)),
            _full(Wp2.shape), _full((1, C)),
            _full(Wa1.shape), _full((1, C)),
            _full(Wa2.shape), _full((1, C)),
            _full(W_out.shape), _full((1, b_out.shape[0])),
        ],
        out_specs=pl.BlockSpec((1, QB, C), lambda qb: (0, qb, 0)),
        out_shape=jax.ShapeDtypeStruct((1, N, C), _f32),
    )

    outs = []
    for b in range(B):
        q, table, knn = prep(pos[b:b + 1], posT[b:b + 1],
                             features[b:b + 1], W_emb, b_emb2, W_q, W_k, W_v)
        g = sc_gather(table.reshape(N, TW), knn.reshape(ROWS))
        outs.append(attn(g.reshape(1, N * K, TW),
                         q, pos[b:b + 1], features[b:b + 1],
                         Wp1, bp12, Wp2b, bp22, Wa1b, ba12, Wa2b, ba22,
                         W_outb, b_out2))
    return jnp.concatenate(outs, axis=0)
